# baseline (speedup 1.0000x reference)
import functools

import jax
import jax.numpy as jnp
from jax import lax
from jax.experimental import pallas as pl
from jax.experimental.pallas import tpu as pltpu
from jax.experimental.pallas import tpu_sc as plsc

_B = 4096
_S = 200
_D = 128
_K = 100
_SPAD = 256
_NVREG = _SPAD // 16
_KPAD = 112
_NEG = -3.0e38
_BB = 128


def _sim_body(cand_ref, beh_ref, out_ref):
    cand = cand_ref[...]
    beh = beh_ref[...]
    bb = cand.shape[0]
    num = jnp.sum(cand[:, None, :] * beh, axis=-1)
    sq = jnp.sum(beh * beh, axis=-1)
    sim = num * lax.rsqrt(jnp.maximum(sq, jnp.float32(1e-30)))
    pad = jnp.full((bb, _SPAD - _S), _NEG, jnp.float32)
    out_ref[...] = jnp.concatenate([sim, pad], axis=1)


def _sim_matrix(candidate_emb, user_behavior_embs, row0, nrows):
    blk0 = row0 // _BB
    return pl.pallas_call(
        _sim_body,
        grid=(nrows // _BB,),
        in_specs=[
            pl.BlockSpec((_BB, _D), lambda i: (blk0 + i, 0)),
            pl.BlockSpec((_BB, _S, _D), lambda i: (blk0 + i, 0, 0)),
        ],
        out_specs=pl.BlockSpec((_BB, _SPAD), lambda i: (i, 0)),
        out_shape=jax.ShapeDtypeStruct((nrows, _SPAD), jnp.float32),
    )(candidate_emb, user_behavior_embs)


def _ce(a, b, desc):
    va, ia = a
    vb, ib = b
    first = (va > vb) | ((va == vb) & (ia < ib))
    if not desc:
        first = jnp.logical_not(first)
    na_ = jnp.where(first, va, vb)
    ni_ = jnp.where(first, ia, ib)
    nb_ = jnp.where(first, vb, va)
    nj_ = jnp.where(first, ib, ia)
    return (na_, ni_), (nb_, nj_)


def _hw_sort(v, i, desc):
    return plsc.sort_key_val(v, i, descending=desc)


def _bitonic_merge(pairs, desc, sort_out=True):
    n = len(pairs)
    h = n // 2
    while h >= 1:
        for i in range(n):
            if (i % (2 * h)) < h:
                pairs[i], pairs[i + h] = _ce(pairs[i], pairs[i + h], desc)
        h //= 2
    if sort_out:
        for i in range(n):
            v, ix = pairs[i]
            v, ix = _hw_sort(v, ix, desc)
            pairs[i] = (v, ix)
    return pairs


def _take(x, perm):
    dnums = lax.GatherDimensionNumbers(
        offset_dims=(), collapsed_slice_dims=(0,), start_index_map=(0,))
    return lax.gather(x, perm[:, None], dnums, slice_sizes=(1,),
                      mode=lax.GatherScatterMode.PROMISE_IN_BOUNDS)


def _stabilize(final):
    lane = lax.iota(jnp.int32, 16)
    even_lane = (lane % 2) == 0
    perm_even = lane ^ 1
    perm_odd = jnp.clip(((lane + 1) ^ 1) - 1, 0, 15)
    lane15 = jnp.full_like(lane, 15)
    lane0 = jnp.zeros_like(lane)
    n = len(final)
    for j in range(n):
        v, ix = final[j]
        pv = _take(v, perm_even)
        pi = _take(ix, perm_even)
        tied = v == pv
        swap = tied & jnp.where(even_lane, ix > pi, ix < pi)
        final[j] = (v, jnp.where(swap, pi, ix))
    vs = [f[0] for f in final]
    ixs = [f[1] for f in final]
    out = []
    for j in range(n):
        v, ix = vs[j], ixs[j]
        pv = _take(v, perm_odd)
        pi = _take(ix, perm_odd)
        if j > 0:
            bv = _take(vs[j - 1], lane15)
            bi = _take(ixs[j - 1], lane15)
            pv = jnp.where(lane == 0, bv, pv)
            pi = jnp.where(lane == 0, bi, pi)
        if j + 1 < n:
            bv = _take(vs[j + 1], lane0)
            bi = _take(ixs[j + 1], lane0)
            pv = jnp.where(lane == 15, bv, pv)
            pi = jnp.where(lane == 15, bi, pi)
        tied = v == pv
        first_of_pair = ~even_lane
        if j == 0:
            valid = lane > 0
        else:
            valid = lane >= 0
        if j + 1 >= n:
            valid = valid & (lane < 15)
        swap = tied & valid & jnp.where(first_of_pair, ix > pi, ix < pi)
        out.append((v, jnp.where(swap, pi, ix)))
    return out


def _topk_vecs(vals):
    pairs = []
    lane = lax.iota(jnp.int32, 16)
    for j in range(_NVREG):
        ix = lane + jnp.int32(16 * j)
        desc = (j % 2) == 0
        v, ix = _hw_sort(vals[j], ix, desc)
        pairs.append((v, ix))
    m = 1
    while m < _NVREG // 2:
        nruns = _NVREG // (2 * m)
        new = []
        for t in range(nruns):
            block = pairs[2 * t * m:(2 * t + 2) * m]
            new.extend(_bitonic_merge(block, desc=(t % 2 == 0)))
        pairs = new
        m *= 2
    h = _NVREG // 2
    for i2 in range(h):
        pairs[i2], pairs[i2 + h] = _ce(pairs[i2], pairs[i2 + h], True)
    top = _bitonic_merge(pairs[:h], desc=True, sort_out=False)
    kv = _KPAD // 16
    final = []
    for j in range(kv):
        v, ix = top[j]
        v, ix = _hw_sort(v, ix, True)
        final.append((v, ix))
    final = _stabilize(final)
    return [f[1] for f in final]


def _topk_row(slab, outs, i):
    vals = [slab[i, pl.ds(16 * j, 16)] for j in range(_NVREG)]
    idx_vecs = _topk_vecs(vals)
    for j in range(len(idx_vecs)):
        outs[i, pl.ds(16 * j, 16)] = idx_vecs[j]


def _topk_sc(sim):
    nb = sim.shape[0]
    info = plsc.get_sparse_core_info()
    nc, ns = info.num_cores, info.num_subcores
    nw = nc * ns
    rpw = nb // nw

    mesh = plsc.VectorSubcoreMesh(core_axis_name="c", subcore_axis_name="s")

    @functools.partial(
        pl.kernel,
        mesh=mesh,
        out_type=jax.ShapeDtypeStruct((nb, _KPAD), jnp.int32),
        scratch_types=[
            pltpu.VMEM((rpw, _SPAD), jnp.float32),
            pltpu.VMEM((rpw, _KPAD), jnp.int32),
        ],
        compiler_params=pltpu.CompilerParams(needs_layout_passes=False),
    )
    def k(sim_hbm, out_hbm, slab, outs):
        wid = lax.axis_index("s") * nc + lax.axis_index("c")
        base = wid * rpw
        pltpu.sync_copy(sim_hbm.at[pl.ds(base, rpw)], slab)

        def row(i, carry):
            _topk_row(slab, outs, i)
            return carry

        lax.fori_loop(0, rpw, row, jnp.int32(0))
        pltpu.sync_copy(outs, out_hbm.at[pl.ds(base, rpw)])

    return k(sim)


_NCHUNK = 4


def kernel(candidate_emb, user_behavior_embs, k):
    b = candidate_emb.shape[0]
    cs = b // _NCHUNK
    sims = [_sim_matrix(candidate_emb, user_behavior_embs, c * cs, cs)
            for c in range(_NCHUNK)]
    outs = [_topk_sc(s) for s in sims]
    padded = jnp.concatenate(outs, axis=0)
    return padded[:, :_K]


# speedup vs baseline: 1.0090x; 1.0090x over previous
import functools

import jax
import jax.numpy as jnp
from jax import lax
from jax.experimental import pallas as pl
from jax.experimental.pallas import tpu as pltpu
from jax.experimental.pallas import tpu_sc as plsc

_B = 4096
_S = 200
_D = 128
_K = 100
_SPAD = 256
_NVREG = _SPAD // 16
_KPAD = 112
_NEG = -3.0e38
_BB = 128


def _sim_body(cand_ref, beh_ref, out_ref):
    cand = cand_ref[...]
    beh = beh_ref[...]
    bb = cand.shape[0]
    num = jnp.sum(cand[:, None, :] * beh, axis=-1)
    sq = jnp.sum(beh * beh, axis=-1)
    sim = num * lax.rsqrt(jnp.maximum(sq, jnp.float32(1e-30)))
    pad = jnp.full((bb, _SPAD - _S), _NEG, jnp.float32)
    out_ref[...] = jnp.concatenate([sim, pad], axis=1)


def _sim_matrix(candidate_emb, user_behavior_embs, row0, nrows):
    blk0 = row0 // _BB
    return pl.pallas_call(
        _sim_body,
        grid=(nrows // _BB,),
        in_specs=[
            pl.BlockSpec((_BB, _D), lambda i: (blk0 + i, 0)),
            pl.BlockSpec((_BB, _S, _D), lambda i: (blk0 + i, 0, 0)),
        ],
        out_specs=pl.BlockSpec((_BB, _SPAD), lambda i: (i, 0)),
        out_shape=jax.ShapeDtypeStruct((nrows, _SPAD), jnp.float32),
    )(candidate_emb, user_behavior_embs)


def _ce(a, b, desc):
    va, ia = a
    vb, ib = b
    first = (va > vb) | ((va == vb) & (ia < ib))
    if not desc:
        first = jnp.logical_not(first)
    na_ = jnp.where(first, va, vb)
    ni_ = jnp.where(first, ia, ib)
    nb_ = jnp.where(first, vb, va)
    nj_ = jnp.where(first, ib, ia)
    return (na_, ni_), (nb_, nj_)


def _hw_sort(v, i, desc):
    return plsc.sort_key_val(v, i, descending=desc)


def _bitonic_merge(pairs, desc, sort_out=True):
    n = len(pairs)
    h = n // 2
    while h >= 1:
        for i in range(n):
            if (i % (2 * h)) < h:
                pairs[i], pairs[i + h] = _ce(pairs[i], pairs[i + h], desc)
        h //= 2
    if sort_out:
        for i in range(n):
            v, ix = pairs[i]
            v, ix = _hw_sort(v, ix, desc)
            pairs[i] = (v, ix)
    return pairs


def _take(x, perm):
    dnums = lax.GatherDimensionNumbers(
        offset_dims=(), collapsed_slice_dims=(0,), start_index_map=(0,))
    return lax.gather(x, perm[:, None], dnums, slice_sizes=(1,),
                      mode=lax.GatherScatterMode.PROMISE_IN_BOUNDS)


def _stabilize(final):
    lane = lax.iota(jnp.int32, 16)
    even_lane = (lane % 2) == 0
    perm_even = lane ^ 1
    perm_odd = jnp.clip(((lane + 1) ^ 1) - 1, 0, 15)
    lane15 = jnp.full_like(lane, 15)
    lane0 = jnp.zeros_like(lane)
    n = len(final)
    for j in range(n):
        v, ix = final[j]
        pv = _take(v, perm_even)
        pi = _take(ix, perm_even)
        tied = v == pv
        swap = tied & jnp.where(even_lane, ix > pi, ix < pi)
        final[j] = (v, jnp.where(swap, pi, ix))
    vs = [f[0] for f in final]
    ixs = [f[1] for f in final]
    out = []
    for j in range(n):
        v, ix = vs[j], ixs[j]
        pv = _take(v, perm_odd)
        pi = _take(ix, perm_odd)
        if j > 0:
            bv = _take(vs[j - 1], lane15)
            bi = _take(ixs[j - 1], lane15)
            pv = jnp.where(lane == 0, bv, pv)
            pi = jnp.where(lane == 0, bi, pi)
        if j + 1 < n:
            bv = _take(vs[j + 1], lane0)
            bi = _take(ixs[j + 1], lane0)
            pv = jnp.where(lane == 15, bv, pv)
            pi = jnp.where(lane == 15, bi, pi)
        tied = v == pv
        first_of_pair = ~even_lane
        if j == 0:
            valid = lane > 0
        else:
            valid = lane >= 0
        if j + 1 >= n:
            valid = valid & (lane < 15)
        swap = tied & valid & jnp.where(first_of_pair, ix > pi, ix < pi)
        out.append((v, jnp.where(swap, pi, ix)))
    return out


def _topk_vecs(vals):
    pairs = []
    lane = lax.iota(jnp.int32, 16)
    for j in range(_NVREG):
        ix = lane + jnp.int32(16 * j)
        desc = (j % 2) == 0
        v, ix = _hw_sort(vals[j], ix, desc)
        pairs.append((v, ix))
    m = 1
    while m < _NVREG // 2:
        nruns = _NVREG // (2 * m)
        new = []
        for t in range(nruns):
            block = pairs[2 * t * m:(2 * t + 2) * m]
            new.extend(_bitonic_merge(block, desc=(t % 2 == 0)))
        pairs = new
        m *= 2
    h = _NVREG // 2
    for i2 in range(h):
        pairs[i2], pairs[i2 + h] = _ce(pairs[i2], pairs[i2 + h], True)
    top = _bitonic_merge(pairs[:h], desc=True, sort_out=False)
    kv = _KPAD // 16
    final = []
    for j in range(kv):
        v, ix = top[j]
        v, ix = _hw_sort(v, ix, True)
        final.append((v, ix))
    final = _stabilize(final)
    return [f[1] for f in final]


def _topk_row(slab, outs, i):
    vals = [slab[i, pl.ds(16 * j, 16)] for j in range(_NVREG)]
    idx_vecs = _topk_vecs(vals)
    for j in range(len(idx_vecs)):
        outs[i, pl.ds(16 * j, 16)] = idx_vecs[j]


def _topk_sc(sim):
    nb = sim.shape[0]
    info = plsc.get_sparse_core_info()
    nc, ns = info.num_cores, info.num_subcores
    nw = nc * ns
    rpw = nb // nw

    mesh = plsc.VectorSubcoreMesh(core_axis_name="c", subcore_axis_name="s")

    @functools.partial(
        pl.kernel,
        mesh=mesh,
        out_type=jax.ShapeDtypeStruct((nb, _KPAD), jnp.int32),
        scratch_types=[
            pltpu.VMEM((rpw, _SPAD), jnp.float32),
            pltpu.VMEM((rpw, _KPAD), jnp.int32),
        ],
        compiler_params=pltpu.CompilerParams(needs_layout_passes=False),
    )
    def k(sim_hbm, out_hbm, slab, outs):
        wid = lax.axis_index("s") * nc + lax.axis_index("c")
        base = wid * rpw
        pltpu.sync_copy(sim_hbm.at[pl.ds(base, rpw)], slab)

        @plsc.parallel_loop(0, rpw, 1, unroll=2)
        def _(i):
            _topk_row(slab, outs, i)
        pltpu.sync_copy(outs, out_hbm.at[pl.ds(base, rpw)])

    return k(sim)


_NCHUNK = 1


def kernel(candidate_emb, user_behavior_embs, k):
    b = candidate_emb.shape[0]
    cs = b // _NCHUNK
    sims = [_sim_matrix(candidate_emb, user_behavior_embs, c * cs, cs)
            for c in range(_NCHUNK)]
    outs = [_topk_sc(s) for s in sims]
    padded = jnp.concatenate(outs, axis=0)
    return padded[:, :_K]
